# Initial kernel scaffold; baseline (speedup 1.0000x reference)
import jax
import jax.numpy as jnp
from jax.experimental import pallas as pl


def kernel(src_node_features, neighbor_node_features, W_agg, b):
    raise NotImplementedError("write your pallas kernel here")


# speedup vs baseline: 1.3187x; 1.3187x over previous
import jax
import jax.numpy as jnp
from jax.experimental import pallas as pl

N = 10000
DEG = 32
D = 128
BLOCK = 400


def _fused_body(src_ref, neigh_ref, w_ref, b_ref, out_ref):
    agg = jnp.sum(neigh_ref[...], axis=1) * (1.0 / DEG)
    h = jnp.dot(agg, w_ref[...], preferred_element_type=jnp.float32)
    h += jnp.dot(src_ref[...], b_ref[...], preferred_element_type=jnp.float32)
    out_ref[...] = jnp.maximum(h, 0.0)


def kernel(src_node_features, neighbor_node_features, W_agg, b):
    grid = N // BLOCK
    return pl.pallas_call(
        _fused_body,
        grid=(grid,),
        in_specs=[
            pl.BlockSpec((BLOCK, D), lambda i: (i, 0)),
            pl.BlockSpec((BLOCK, DEG, D), lambda i: (i, 0, 0)),
            pl.BlockSpec((D, D), lambda i: (0, 0)),
            pl.BlockSpec((D, D), lambda i: (0, 0)),
        ],
        out_specs=pl.BlockSpec((BLOCK, D), lambda i: (i, 0)),
        out_shape=jax.ShapeDtypeStruct((N, D), jnp.float32),
    )(src_node_features, neighbor_node_features, W_agg, b)
